# baseline (speedup 1.0000x reference)
import functools

import jax
import jax.numpy as jnp
from jax.experimental import pallas as pl
from jax.experimental.pallas import tpu as pltpu

N = 10000
F = 256


def _layer_kernel(adj_ref, x_ref, w_ref, b_ref, o_ref, *, relu):
    agg = jnp.dot(adj_ref[...], x_ref[...],
                  preferred_element_type=jnp.float32)
    acc = jnp.dot(agg, w_ref[...], preferred_element_type=jnp.float32)
    acc = acc + b_ref[...]
    if relu:
        acc = jnp.maximum(acc, 0.0)
    o_ref[...] = acc


def _layer(adj, x, w, b, relu):
    br = 200
    return pl.pallas_call(
        functools.partial(_layer_kernel, relu=relu),
        grid=(N // br,),
        in_specs=[
            pl.BlockSpec((br, N), lambda i: (i, 0)),
            pl.BlockSpec((N, F), lambda i: (0, 0)),
            pl.BlockSpec((F, F), lambda i: (0, 0)),
            pl.BlockSpec((1, F), lambda i: (0, 0)),
        ],
        out_specs=pl.BlockSpec((br, F), lambda i: (i, 0)),
        out_shape=jax.ShapeDtypeStruct((N, F), jnp.float32),
        compiler_params=pltpu.CompilerParams(
            dimension_semantics=("arbitrary",),
        ),
    )(adj, x, w, b)


def kernel(x, adj, W1, b1, W2, b2):
    h = _layer(adj, x, W1, b1.reshape(1, F), relu=True)
    return _layer(adj, h, W2, b2.reshape(1, F), relu=False)


# speedup vs baseline: 1.0233x; 1.0233x over previous
import functools

import jax
import jax.numpy as jnp
from jax.experimental import pallas as pl
from jax.experimental.pallas import tpu as pltpu

N = 10000
F = 256


def _layer_kernel(adj_ref, x_ref, w_ref, b_ref, o_ref, *, relu):
    agg = jnp.dot(adj_ref[...], x_ref[...],
                  preferred_element_type=jnp.float32)
    acc = jnp.dot(agg, w_ref[...], preferred_element_type=jnp.float32)
    acc = acc + b_ref[...]
    if relu:
        acc = jnp.maximum(acc, 0.0)
    o_ref[...] = acc


def _layer(adj, x, w, b, relu):
    br = 400
    return pl.pallas_call(
        functools.partial(_layer_kernel, relu=relu),
        grid=(N // br,),
        in_specs=[
            pl.BlockSpec((br, N), lambda i: (i, 0)),
            pl.BlockSpec((N, F), lambda i: (0, 0)),
            pl.BlockSpec((F, F), lambda i: (0, 0)),
            pl.BlockSpec((1, F), lambda i: (0, 0)),
        ],
        out_specs=pl.BlockSpec((br, F), lambda i: (i, 0)),
        out_shape=jax.ShapeDtypeStruct((N, F), jnp.float32),
        compiler_params=pltpu.CompilerParams(
            dimension_semantics=("arbitrary",),
        ),
    )(adj, x, w, b)


def kernel(x, adj, W1, b1, W2, b2):
    h = _layer(adj, x, W1, b1.reshape(1, F), relu=True)
    return _layer(adj, h, W2, b2.reshape(1, F), relu=False)
